# Initial kernel scaffold; baseline (speedup 1.0000x reference)
import jax
import jax.numpy as jnp
from jax.experimental import pallas as pl


def kernel(adj0_indices, adj0_values, adj1_indices, adj1_values, adj2_indices, adj2_values, x_dv, W0_vh, W1_vh, W2_vh, fc_w, fc_b):
    raise NotImplementedError("write your pallas kernel here")


# speedup vs baseline: 4.5487x; 4.5487x over previous
import functools

import jax
import jax.numpy as jnp
from jax import lax
from jax.experimental import pallas as pl
from jax.experimental.pallas import tpu as pltpu
from jax.experimental.pallas import tpu_sc as plsc

V = 16384
HID = 64
OUTF = 64

NC = 2
NS = 16
NW = NC * NS
C = 256
ROWS_PER_TILE = V // NS

_GD = lax.GatherDimensionNumbers(
    offset_dims=(), collapsed_slice_dims=(0,), start_index_map=(0,))


def _splat(vv, j):
    idx = jnp.full((16, 1), j, dtype=jnp.int32)
    return lax.gather(vv, idx, _GD, (1,),
                      mode=lax.GatherScatterMode.PROMISE_IN_BOUNDS)


def _sc_spmm(rows2d, cols2d, vals, w_all, zeros_vh, nch):
    mesh = plsc.VectorSubcoreMesh(core_axis_name="c", subcore_axis_name="s")
    nj = C // 128

    @functools.partial(
        pl.kernel,
        out_type=jax.ShapeDtypeStruct((NC, V, HID), jnp.float32),
        mesh=mesh,
        scratch_types=[
            pltpu.VMEM((nj, 128), jnp.int32),
            pltpu.VMEM((nj, 128), jnp.int32),
            pltpu.VMEM((nj, 128), jnp.int32),
            pltpu.VMEM((nj, 128), jnp.int32),
            pltpu.VMEM((nj, 128), jnp.int32),
            pltpu.VMEM((nj, 128), jnp.int32),
            pltpu.VMEM((C,), jnp.float32),
            pltpu.VMEM((C,), jnp.float32),
            pltpu.VMEM((C,), jnp.float32),
            pltpu.VMEM((C, HID), jnp.float32),
            pltpu.VMEM((C, HID), jnp.float32),
            pltpu.VMEM_SHARED((V, HID), jnp.float32),
            pltpu.SemaphoreType.DMA,
            pltpu.SemaphoreType.DMA,
            pltpu.SemaphoreType.DMA,
            pltpu.SemaphoreType.DMA,
            pltpu.SemaphoreType.DMA,
            pltpu.SemaphoreType.DMA,
            pltpu.SemaphoreType.DMA,
        ],
        compiler_params=pltpu.CompilerParams(use_tc_tiling_on_sc=False),
    )
    def k(rows_hbm, cols_hbm, vals_hbm, w_hbm, z_hbm, out_hbm,
          r0b, r1b, r2b, c0b, c1b, c2b, v0b, v1b, v2b, g0b, g1b, h_sh,
          is0, is1, is2, gs0, gs1, ss0, ss1):
        cid = lax.axis_index("c")
        sid = lax.axis_index("s")
        wid = sid * NC + cid
        ebase = wid * (nch * C)
        rbase = wid * (nch * nj)

        rows_b = (r0b, r1b, r2b)
        cols_b = (c0b, c1b, c2b)
        vals_b = (v0b, v1b, v2b)
        gath_b = (g0b, g1b)
        isem = (is0, is1, is2)
        gsem = (gs0, gs1)
        ssem = (ss0, ss1)

        pltpu.sync_copy(z_hbm.at[pl.ds(sid * ROWS_PER_TILE, ROWS_PER_TILE)],
                        h_sh.at[pl.ds(sid * ROWS_PER_TILE, ROWS_PER_TILE)])
        plsc.subcore_barrier()

        def issue_idx(kc, t):
            r0 = rbase + kc * nj
            e0 = ebase + kc * C
            pltpu.async_copy(rows_hbm.at[pl.ds(r0, nj)], rows_b[t], isem[t])
            pltpu.async_copy(cols_hbm.at[pl.ds(r0, nj)], cols_b[t], isem[t])
            pltpu.async_copy(vals_hbm.at[pl.ds(e0, C)], vals_b[t], isem[t])

        def wait_idx(t):
            pltpu.make_async_copy(rows_hbm.at[pl.ds(0, nj)], rows_b[t], isem[t]).wait()
            pltpu.make_async_copy(cols_hbm.at[pl.ds(0, nj)], cols_b[t], isem[t]).wait()
            pltpu.make_async_copy(vals_hbm.at[pl.ds(0, C)], vals_b[t], isem[t]).wait()

        def issue_gather(t, b):
            for j in range(nj):
                pltpu.async_copy(w_hbm.at[cols_b[t].at[j]],
                                 gath_b[b].at[pl.ds(j * 128, 128)], gsem[b])

        def wait_gather(t, b):
            for j in range(nj):
                pltpu.make_async_copy(w_hbm.at[cols_b[t].at[j]],
                                      gath_b[b].at[pl.ds(j * 128, 128)],
                                      gsem[b]).wait()

        def issue_scatter(t, b):
            for j in range(nj):
                pltpu.async_copy(gath_b[b].at[pl.ds(j * 128, 128)],
                                 h_sh.at[rows_b[t].at[j]], ssem[b], add=True)

        def wait_scatter(t, b):
            for j in range(nj):
                pltpu.make_async_copy(gath_b[b].at[pl.ds(j * 128, 128)],
                                      h_sh.at[rows_b[t].at[j]], ssem[b]).wait()

        def scale(t, b):
            vb = vals_b[t]
            gb = gath_b[b]

            def g_body(g, carry):
                vv = vb[pl.ds(g * 16, 16)]
                for j in range(16):
                    sp = _splat(vv, j)
                    e = g * 16 + j
                    for q in range(HID // 16):
                        sl = pl.ds(q * 16, 16)
                        gb[e, sl] = gb[e, sl] * sp
                return carry

            lax.fori_loop(0, C // 16, g_body, 0)

        def chunk(kc, carry):
            issue_idx(kc, 0)
            wait_idx(0)
            issue_gather(0, 0)
            wait_gather(0, 0)
            scale(0, 0)
            issue_scatter(0, 0)
            wait_scatter(0, 0)
            return carry

        lax.fori_loop(0, nch, chunk, 0)
        plsc.subcore_barrier()

        pltpu.sync_copy(h_sh.at[pl.ds(sid * ROWS_PER_TILE, ROWS_PER_TILE)],
                        out_hbm.at[cid, pl.ds(sid * ROWS_PER_TILE, ROWS_PER_TILE)])

    return k(rows2d, cols2d, vals, w_all, zeros_vh)


def _tc_project(x2, h0, h1, fc_w, fc_b2):
    M = x2.shape[0]
    KB = 2048
    nk = V // KB

    def body(x_ref, h0_ref, h1_ref, w_ref, b_ref, o_ref, acc_ref):
        kc = pl.program_id(0)

        @pl.when(kc == 0)
        def _():
            acc_ref[...] = jnp.zeros_like(acc_ref)

        h = h0_ref[...] + h1_ref[...]
        acc_ref[...] += jnp.dot(x_ref[...], h,
                                preferred_element_type=jnp.float32)

        @pl.when(kc == nk - 1)
        def _():
            o_ref[...] = (jnp.dot(acc_ref[...], w_ref[...],
                                  preferred_element_type=jnp.float32)
                          + b_ref[...])

    return pl.pallas_call(
        body,
        grid=(nk,),
        in_specs=[
            pl.BlockSpec((M, KB), lambda kc: (0, kc)),
            pl.BlockSpec((KB, HID), lambda kc: (kc, 0)),
            pl.BlockSpec((KB, HID), lambda kc: (kc, 0)),
            pl.BlockSpec((HID, OUTF), lambda kc: (0, 0)),
            pl.BlockSpec((1, OUTF), lambda kc: (0, 0)),
        ],
        out_specs=pl.BlockSpec((M, OUTF), lambda kc: (0, 0)),
        out_shape=jax.ShapeDtypeStruct((M, OUTF), jnp.float32),
        scratch_shapes=[pltpu.VMEM((M, OUTF), jnp.float32)],
        compiler_params=pltpu.CompilerParams(
            dimension_semantics=("arbitrary",)),
    )(x2, h0, h1, fc_w, fc_b2)


def kernel(adj0_indices, adj0_values, adj1_indices, adj1_values,
           adj2_indices, adj2_values, x_dv, W0_vh, W1_vh, W2_vh, fc_w, fc_b):
    rows = jnp.concatenate(
        [adj0_indices[0], adj1_indices[0], adj2_indices[0]])
    cols = jnp.concatenate(
        [adj0_indices[1], adj1_indices[1] + V, adj2_indices[1] + 2 * V])
    vals = jnp.concatenate([adj0_values, adj1_values, adj2_values])

    total = rows.shape[0]
    wave = NW * C
    nch = -(-total // wave)
    ne = nch * wave
    pad = ne - total
    rows = jnp.pad(rows, (0, pad))
    cols = jnp.pad(cols, (0, pad))
    vals = jnp.pad(vals, (0, pad))

    w_all = jnp.concatenate([W0_vh, W1_vh, W2_vh], axis=0)
    zeros_vh = jnp.zeros((V, HID), jnp.float32)

    hp = _sc_spmm(rows.reshape(-1, 128), cols.reshape(-1, 128), vals,
                  w_all, zeros_vh, nch)

    b, d, _ = x_dv.shape
    x2 = x_dv.reshape(b * d, V)
    out = _tc_project(x2, hp[0], hp[1], fc_w, fc_b.reshape(1, OUTF))
    return out.reshape(b, d, OUTF)


# speedup vs baseline: 5.4259x; 1.1929x over previous
import functools

import jax
import jax.numpy as jnp
from jax import lax
from jax.experimental import pallas as pl
from jax.experimental.pallas import tpu as pltpu
from jax.experimental.pallas import tpu_sc as plsc

V = 16384
HID = 64
OUTF = 64

NC = 2
NS = 16
NW = NC * NS
C = 256
ROWS_PER_TILE = V // NS

_GD = lax.GatherDimensionNumbers(
    offset_dims=(), collapsed_slice_dims=(0,), start_index_map=(0,))


def _splat(vv, j):
    idx = jnp.full((16, 1), j, dtype=jnp.int32)
    return lax.gather(vv, idx, _GD, (1,),
                      mode=lax.GatherScatterMode.PROMISE_IN_BOUNDS)


def _sc_spmm(rows2d, cols2d, vals, w_all, zeros_vh, nch):
    mesh = plsc.VectorSubcoreMesh(core_axis_name="c", subcore_axis_name="s")
    nj = C // 128

    @functools.partial(
        pl.kernel,
        out_type=jax.ShapeDtypeStruct((NC, V, HID), jnp.float32),
        mesh=mesh,
        scratch_types=[
            pltpu.VMEM((nj, 128), jnp.int32),
            pltpu.VMEM((nj, 128), jnp.int32),
            pltpu.VMEM((nj, 128), jnp.int32),
            pltpu.VMEM((nj, 128), jnp.int32),
            pltpu.VMEM((nj, 128), jnp.int32),
            pltpu.VMEM((nj, 128), jnp.int32),
            pltpu.VMEM((C,), jnp.float32),
            pltpu.VMEM((C,), jnp.float32),
            pltpu.VMEM((C,), jnp.float32),
            pltpu.VMEM((C, HID), jnp.float32),
            pltpu.VMEM((C, HID), jnp.float32),
            pltpu.VMEM_SHARED((V, HID), jnp.float32),
            pltpu.SemaphoreType.DMA,
            pltpu.SemaphoreType.DMA,
            pltpu.SemaphoreType.DMA,
            pltpu.SemaphoreType.DMA,
            pltpu.SemaphoreType.DMA,
            pltpu.SemaphoreType.DMA,
            pltpu.SemaphoreType.DMA,
        ],
        compiler_params=pltpu.CompilerParams(use_tc_tiling_on_sc=False),
    )
    def k(rows_hbm, cols_hbm, vals_hbm, w_hbm, z_hbm, out_hbm,
          r0b, r1b, r2b, c0b, c1b, c2b, v0b, v1b, v2b, g0b, g1b, h_sh,
          is0, is1, is2, gs0, gs1, ss0, ss1):
        cid = lax.axis_index("c")
        sid = lax.axis_index("s")
        wid = sid * NC + cid
        ebase = wid * (nch * C)
        rbase = wid * (nch * nj)

        rows_b = (r0b, r1b, r2b)
        cols_b = (c0b, c1b, c2b)
        vals_b = (v0b, v1b, v2b)
        gath_b = (g0b, g1b)
        isem = (is0, is1, is2)
        gsem = (gs0, gs1)
        ssem = (ss0, ss1)

        pltpu.sync_copy(z_hbm.at[pl.ds(sid * ROWS_PER_TILE, ROWS_PER_TILE)],
                        h_sh.at[pl.ds(sid * ROWS_PER_TILE, ROWS_PER_TILE)])
        plsc.subcore_barrier()

        def issue_idx(kc, t):
            r0 = rbase + kc * nj
            e0 = ebase + kc * C
            pltpu.async_copy(rows_hbm.at[pl.ds(r0, nj)], rows_b[t], isem[t])
            pltpu.async_copy(cols_hbm.at[pl.ds(r0, nj)], cols_b[t], isem[t])
            pltpu.async_copy(vals_hbm.at[pl.ds(e0, C)], vals_b[t], isem[t])

        def wait_idx(t):
            pltpu.make_async_copy(rows_hbm.at[pl.ds(0, nj)], rows_b[t], isem[t]).wait()
            pltpu.make_async_copy(cols_hbm.at[pl.ds(0, nj)], cols_b[t], isem[t]).wait()
            pltpu.make_async_copy(vals_hbm.at[pl.ds(0, C)], vals_b[t], isem[t]).wait()

        def issue_gather(t, b):
            for j in range(nj):
                pltpu.async_copy(w_hbm.at[cols_b[t].at[j]],
                                 gath_b[b].at[pl.ds(j * 128, 128)], gsem[b])

        def wait_gather(t, b):
            for j in range(nj):
                pltpu.make_async_copy(w_hbm.at[cols_b[t].at[j]],
                                      gath_b[b].at[pl.ds(j * 128, 128)],
                                      gsem[b]).wait()

        def issue_scatter(t, b):
            for j in range(nj):
                pltpu.async_copy(gath_b[b].at[pl.ds(j * 128, 128)],
                                 h_sh.at[rows_b[t].at[j]], ssem[b], add=True)

        def wait_scatter(t, b):
            for j in range(nj):
                pltpu.make_async_copy(gath_b[b].at[pl.ds(j * 128, 128)],
                                      h_sh.at[rows_b[t].at[j]], ssem[b]).wait()

        def scale(t, b):
            vb = vals_b[t]
            gb = gath_b[b]

            def g_body(g, carry):
                vv = vb[pl.ds(g * 16, 16)]
                for j in range(16):
                    sp = _splat(vv, j)
                    e = g * 16 + j
                    for q in range(HID // 16):
                        sl = pl.ds(q * 16, 16)
                        gb[e, sl] = gb[e, sl] * sp
                return carry

            lax.fori_loop(0, C // 16, g_body, 0)

        def half(kc, u, first=False, tail=0):
            b, t = u % 2, u % 3
            nb = 1 - b
            wait_gather(t, b)
            if not first:
                wait_scatter((u - 1) % 3, nb)
            if tail < 2:
                wait_idx((u + 1) % 3)
                issue_gather((u + 1) % 3, nb)
            if tail < 1:
                issue_idx(kc + 2, (u + 2) % 3)
            scale(t, b)
            issue_scatter(t, b)

        issue_idx(0, 0)
        issue_idx(1, 1)
        wait_idx(0)
        issue_gather(0, 0)
        half(0, 0, first=True)
        half(1, 1)

        def six(kk, carry):
            k0 = 2 + kk * 6
            for u in range(6):
                half(k0 + u, 2 + u)
            return carry

        lax.fori_loop(0, (nch - 4) // 6, six, 0)

        half(nch - 2, (nch - 2) % 6, tail=1)
        half(nch - 1, (nch - 1) % 6, tail=2)
        wait_scatter((nch - 1) % 3, (nch - 1) % 2)
        plsc.subcore_barrier()

        pltpu.sync_copy(h_sh.at[pl.ds(sid * ROWS_PER_TILE, ROWS_PER_TILE)],
                        out_hbm.at[cid, pl.ds(sid * ROWS_PER_TILE, ROWS_PER_TILE)])

    return k(rows2d, cols2d, vals, w_all, zeros_vh)


def _tc_project(x2, h0, h1, fc_w, fc_b2):
    M = x2.shape[0]
    KB = 2048
    nk = V // KB

    def body(x_ref, h0_ref, h1_ref, w_ref, b_ref, o_ref, acc_ref):
        kc = pl.program_id(0)

        @pl.when(kc == 0)
        def _():
            acc_ref[...] = jnp.zeros_like(acc_ref)

        h = h0_ref[...] + h1_ref[...]
        acc_ref[...] += jnp.dot(x_ref[...], h,
                                preferred_element_type=jnp.float32)

        @pl.when(kc == nk - 1)
        def _():
            o_ref[...] = (jnp.dot(acc_ref[...], w_ref[...],
                                  preferred_element_type=jnp.float32)
                          + b_ref[...])

    return pl.pallas_call(
        body,
        grid=(nk,),
        in_specs=[
            pl.BlockSpec((M, KB), lambda kc: (0, kc)),
            pl.BlockSpec((KB, HID), lambda kc: (kc, 0)),
            pl.BlockSpec((KB, HID), lambda kc: (kc, 0)),
            pl.BlockSpec((HID, OUTF), lambda kc: (0, 0)),
            pl.BlockSpec((1, OUTF), lambda kc: (0, 0)),
        ],
        out_specs=pl.BlockSpec((M, OUTF), lambda kc: (0, 0)),
        out_shape=jax.ShapeDtypeStruct((M, OUTF), jnp.float32),
        scratch_shapes=[pltpu.VMEM((M, OUTF), jnp.float32)],
        compiler_params=pltpu.CompilerParams(
            dimension_semantics=("arbitrary",)),
    )(x2, h0, h1, fc_w, fc_b2)


def kernel(adj0_indices, adj0_values, adj1_indices, adj1_values,
           adj2_indices, adj2_values, x_dv, W0_vh, W1_vh, W2_vh, fc_w, fc_b):
    rows = jnp.concatenate(
        [adj0_indices[0], adj1_indices[0], adj2_indices[0]])
    cols = jnp.concatenate(
        [adj0_indices[1], adj1_indices[1] + V, adj2_indices[1] + 2 * V])
    vals = jnp.concatenate([adj0_values, adj1_values, adj2_values])

    total = rows.shape[0]
    wave = NW * C
    nch = -(-total // wave)
    nch += (4 - nch) % 6
    ne = nch * wave
    pad = ne - total
    rows = jnp.pad(rows, (0, pad))
    cols = jnp.pad(cols, (0, pad))
    vals = jnp.pad(vals, (0, pad))

    w_all = jnp.concatenate([W0_vh, W1_vh, W2_vh], axis=0)
    zeros_vh = jnp.zeros((V, HID), jnp.float32)

    hp = _sc_spmm(rows.reshape(-1, 128), cols.reshape(-1, 128), vals,
                  w_all, zeros_vh, nch)

    b, d, _ = x_dv.shape
    x2 = x_dv.reshape(b * d, V)
    out = _tc_project(x2, hp[0], hp[1], fc_w, fc_b.reshape(1, OUTF))
    return out.reshape(b, d, OUTF)


# speedup vs baseline: 7.5660x; 1.3944x over previous
import functools

import jax
import jax.numpy as jnp
from jax import lax
from jax.experimental import pallas as pl
from jax.experimental.pallas import tpu as pltpu
from jax.experimental.pallas import tpu_sc as plsc

V = 16384
HID = 64
OUTF = 64

NC = 2
NS = 16
NW = NC * NS
C = 128
DI = 8
DG = 4
DS = 2
ROWS_PER_TILE = V // NS

_GD = lax.GatherDimensionNumbers(
    offset_dims=(), collapsed_slice_dims=(0,), start_index_map=(0,))


def _splat(vv, j):
    idx = jnp.full((16, 1), j, dtype=jnp.int32)
    return lax.gather(vv, idx, _GD, (1,),
                      mode=lax.GatherScatterMode.PROMISE_IN_BOUNDS)


def _sc_spmm(pk, vals, w_all, zeros_vh, nch):
    mesh = plsc.VectorSubcoreMesh(core_axis_name="c", subcore_axis_name="s")

    @functools.partial(
        pl.kernel,
        out_type=jax.ShapeDtypeStruct((NC, V, HID), jnp.float32),
        mesh=mesh,
        scratch_types=(
            [pltpu.VMEM((2, C), jnp.int32) for _ in range(DI)]
            + [pltpu.VMEM((C,), jnp.float32) for _ in range(DI)]
            + [pltpu.VMEM((C, HID), jnp.float32) for _ in range(DG)]
            + [pltpu.VMEM((C, HID), jnp.float32) for _ in range(DS)]
            + [pltpu.VMEM_SHARED((V, HID), jnp.float32)]
            + [pltpu.SemaphoreType.DMA for _ in range(DI + DG + DS)]
        ),
        compiler_params=pltpu.CompilerParams(use_tc_tiling_on_sc=False),
    )
    def k(pk_hbm, vals_hbm, w_hbm, z_hbm, out_hbm,
          p0, p1, p2, p3, p4, p5, p6, p7,
          v0, v1, v2, v3, v4, v5, v6, v7,
          g0, g1, g2, g3, s0, s1, h_sh,
          ip0, ip1, ip2, ip3, ip4, ip5, ip6, ip7, ig0, ig1, ig2, ig3,
          is0, is1):
        cid = lax.axis_index("c")
        sid = lax.axis_index("s")
        wid = sid * NC + cid
        cbase = wid * nch

        pk_b = (p0, p1, p2, p3, p4, p5, p6, p7)
        vals_b = (v0, v1, v2, v3, v4, v5, v6, v7)
        gath_b = (g0, g1, g2, g3)
        scl_b = (s0, s1)
        isem = (ip0, ip1, ip2, ip3, ip4, ip5, ip6, ip7)
        gsem = (ig0, ig1, ig2, ig3)
        ssem = (is0, is1)

        pltpu.sync_copy(z_hbm.at[pl.ds(sid * ROWS_PER_TILE, ROWS_PER_TILE)],
                        h_sh.at[pl.ds(sid * ROWS_PER_TILE, ROWS_PER_TILE)])
        plsc.subcore_barrier()

        def issue_idx(kc, t):
            pltpu.async_copy(pk_hbm.at[cbase + kc], pk_b[t], isem[t])
            pltpu.async_copy(vals_hbm.at[pl.ds((cbase + kc) * C, C)],
                             vals_b[t], isem[t])

        def wait_idx(t):
            pltpu.make_async_copy(pk_hbm.at[0], pk_b[t], isem[t]).wait()
            pltpu.make_async_copy(vals_hbm.at[pl.ds(0, C)], vals_b[t],
                                  isem[t]).wait()

        def issue_gather(t, g):
            pltpu.async_copy(w_hbm.at[pk_b[t].at[1]], gath_b[g], gsem[g])

        def wait_gather(t, g):
            pltpu.make_async_copy(w_hbm.at[pk_b[t].at[1]], gath_b[g],
                                  gsem[g]).wait()

        def issue_scatter(t, b):
            pltpu.async_copy(scl_b[b], h_sh.at[pk_b[t].at[0]], ssem[b],
                             add=True)

        def wait_scatter(t, b):
            pltpu.make_async_copy(scl_b[b], h_sh.at[pk_b[t].at[0]],
                                  ssem[b]).wait()

        def scale(t, g, b):
            vb = vals_b[t]
            gb = gath_b[g]
            sb = scl_b[b]

            def g_body(gi, carry):
                vv = vb[pl.ds(gi * 16, 16)]
                for j in range(16):
                    sp = _splat(vv, j)
                    e = gi * 16 + j
                    for q in range(HID // 16):
                        sl = pl.ds(q * 16, 16)
                        sb[e, sl] = gb[e, sl] * sp
                return carry

            lax.fori_loop(0, C // 16, g_body, 0)

        def half(kc, u, first=False, steady=False):
            t, g, b = u % DI, u % DG, u % DS
            wait_gather(t, g)
            if not first:
                wait_scatter((u - 1) % DI, (u - 1) % DS)
            if steady or kc + 5 < nch:
                issue_idx(kc + 5, (u + 5) % DI)
            if steady or kc + 3 < nch:
                wait_idx((u + 3) % DI)
                issue_gather((u + 3) % DI, (u + 3) % DG)
            scale(t, g, b)
            issue_scatter(t, b)

        for i in range(5):
            issue_idx(i, i)
        for i in range(3):
            wait_idx(i)
            issue_gather(i, i)
        half(0, 0, first=True)
        half(1, 1)

        def eight(kk, carry):
            k0 = 2 + kk * DI
            for u in range(DI):
                half(k0 + u, 2 + u, steady=True)
            return carry

        lax.fori_loop(0, (nch - 7) // DI, eight, 0)

        for kc in range(nch - 5, nch):
            half(kc, kc % DI)
        wait_scatter((nch - 1) % DI, (nch - 1) % DS)
        plsc.subcore_barrier()

        pltpu.sync_copy(h_sh.at[pl.ds(sid * ROWS_PER_TILE, ROWS_PER_TILE)],
                        out_hbm.at[cid, pl.ds(sid * ROWS_PER_TILE, ROWS_PER_TILE)])

    return k(pk, vals, w_all, zeros_vh)


def _tc_project(x2, h0, h1, fc_w, fc_b2):
    M = x2.shape[0]
    KB = 2048
    nk = V // KB

    def body(x_ref, h0_ref, h1_ref, w_ref, b_ref, o_ref, acc_ref):
        kc = pl.program_id(0)

        @pl.when(kc == 0)
        def _():
            acc_ref[...] = jnp.zeros_like(acc_ref)

        h = h0_ref[...] + h1_ref[...]
        acc_ref[...] += jnp.dot(x_ref[...], h,
                                preferred_element_type=jnp.float32)

        @pl.when(kc == nk - 1)
        def _():
            o_ref[...] = (jnp.dot(acc_ref[...], w_ref[...],
                                  preferred_element_type=jnp.float32)
                          + b_ref[...])

    return pl.pallas_call(
        body,
        grid=(nk,),
        in_specs=[
            pl.BlockSpec((M, KB), lambda kc: (0, kc)),
            pl.BlockSpec((KB, HID), lambda kc: (kc, 0)),
            pl.BlockSpec((KB, HID), lambda kc: (kc, 0)),
            pl.BlockSpec((HID, OUTF), lambda kc: (0, 0)),
            pl.BlockSpec((1, OUTF), lambda kc: (0, 0)),
        ],
        out_specs=pl.BlockSpec((M, OUTF), lambda kc: (0, 0)),
        out_shape=jax.ShapeDtypeStruct((M, OUTF), jnp.float32),
        scratch_shapes=[pltpu.VMEM((M, OUTF), jnp.float32)],
        compiler_params=pltpu.CompilerParams(
            dimension_semantics=("arbitrary",)),
    )(x2, h0, h1, fc_w, fc_b2)


def kernel(adj0_indices, adj0_values, adj1_indices, adj1_values,
           adj2_indices, adj2_values, x_dv, W0_vh, W1_vh, W2_vh, fc_w, fc_b):
    rows = jnp.concatenate(
        [adj0_indices[0], adj1_indices[0], adj2_indices[0]])
    cols = jnp.concatenate(
        [adj0_indices[1], adj1_indices[1] + V, adj2_indices[1] + 2 * V])
    vals = jnp.concatenate([adj0_values, adj1_values, adj2_values])

    total = rows.shape[0]
    wave = NW * C
    nch = -(-total // wave)
    nch += (7 - nch) % 8
    ne = nch * wave
    pad = ne - total
    rows = jnp.pad(rows, (0, pad))
    cols = jnp.pad(cols, (0, pad))
    vals = jnp.pad(vals, (0, pad))

    pk = jnp.stack([rows.reshape(-1, C), cols.reshape(-1, C)], axis=1)

    w_all = jnp.concatenate([W0_vh, W1_vh, W2_vh], axis=0)
    zeros_vh = jnp.zeros((V, HID), jnp.float32)

    hp = _sc_spmm(pk, vals, w_all, zeros_vh, nch)

    b, d, _ = x_dv.shape
    x2 = x_dv.reshape(b * d, V)
    out = _tc_project(x2, hp[0], hp[1], fc_w, fc_b.reshape(1, OUTF))
    return out.reshape(b, d, OUTF)
